# baseline (speedup 1.0000x reference)
import jax
import jax.numpy as jnp
from jax.experimental import pallas as pl
from jax.experimental.pallas import tpu as pltpu

N_NODES = 19719
N_SAMPLES = 313
W_ROWS = 72
N_PAD = 19776
D = 128
EPS = 1e-5
LN2 = 0.6931471805599453
LOG2E = 1.4426950408889634


def _proj_kernel(x_ref, c_ref, w_ref, e_ref, b_ref, o_ref):
    acc = jnp.dot(x_ref[...], w_ref[...], preferred_element_type=jnp.float32)
    c = c_ref[...]
    acc = acc + c[:, 0:1] * e_ref[0:1, :]
    acc = acc + c[:, 1:2] * e_ref[1:2, :]
    o_ref[...] = acc + b_ref[...]


def _proj(x, centers, Wcat, Ecat, bcat):
    R = 512
    return pl.pallas_call(
        _proj_kernel,
        grid=(pl.cdiv(N_PAD, R),),
        in_specs=[
            pl.BlockSpec((R, D), lambda i: (i, 0)),
            pl.BlockSpec((R, 2), lambda i: (i, 0)),
            pl.BlockSpec((D, 4 * D), lambda i: (0, 0)),
            pl.BlockSpec((2, 4 * D), lambda i: (0, 0)),
            pl.BlockSpec((1, 4 * D), lambda i: (0, 0)),
        ],
        out_specs=pl.BlockSpec((R, 4 * D), lambda i: (i, 0)),
        out_shape=jax.ShapeDtypeStruct((N_PAD, 4 * D), jnp.float32),
        compiler_params=pltpu.CompilerParams(
            dimension_semantics=("arbitrary",)),
    )(x, centers, Wcat, Ecat, bcat)


def _pair_kernel(off_ref, cnt_ref, pq_ref, out_ref, s1_ref, s2_ref,
                 bufs, sems):
    k = pl.program_id(0)

    @pl.when(k == 0)
    def _init():
        out_ref[...] = jnp.zeros_like(out_ref)
        s1_ref[...] = jnp.zeros_like(s1_ref)
        s2_ref[...] = jnp.zeros_like(s2_ref)
        o8 = pl.multiple_of((off_ref[0] // 8) * 8, 8)
        pltpu.make_async_copy(
            pq_ref.at[pl.ds(o8, W_ROWS)], bufs.at[0], sems.at[0]).start()

    slot = jax.lax.rem(k, 2)
    nxt = jax.lax.rem(k + 1, 2)
    off = off_ref[k]
    n = cnt_ref[k]
    off8 = pl.multiple_of((off // 8) * 8, 8)
    rem = off - off8

    pltpu.make_async_copy(
        pq_ref.at[pl.ds(off8, W_ROWS)], bufs.at[slot], sems.at[slot]).wait()

    @pl.when(k + 1 < N_SAMPLES)
    def _prefetch():
        o8 = pl.multiple_of((off_ref[k + 1] // 8) * 8, 8)
        pltpu.make_async_copy(
            pq_ref.at[pl.ds(o8, W_ROWS)], bufs.at[nxt], sems.at[nxt]).start()

    pq_vmem = bufs.at[slot]

    rows_w = jax.lax.broadcasted_iota(jnp.int32, (W_ROWS, 1), 0)
    q_ok = (rows_w >= rem) & (rows_w < rem + n)
    pq_vmem[:, D:2 * D] = jnp.where(
        q_ok, pq_vmem[:, D:2 * D], jnp.float32(-1e30))
    pq_vmem[:, 3 * D:4 * D] = jnp.where(
        q_ok, pq_vmem[:, 3 * D:4 * D], jnp.float32(-1e30))

    n_tiles = (rem + n + 7) // 8

    def _msg(af, as_):
        return (1.0 + jnp.tanh(af)) * jnp.log2(1.0 + jnp.exp2(as_))

    def d_body(dt, carry):
        p1, p2 = carry
        db = pl.multiple_of(dt * 8, 8)
        pf_t = pq_vmem[pl.ds(db, 8), 0:D]
        ps_t = pq_vmem[pl.ds(db, 8), 2 * D:3 * D]

        def s_body(st, accs):
            a0, a1 = accs
            sb = pl.multiple_of(st * 8, 8)
            qf_t = pq_vmem[pl.ds(sb, 8), D:2 * D]
            qs_t = pq_vmem[pl.ds(sb, 8), 3 * D:4 * D]
            for j in range(0, 8, 2):
                a0 = a0 + _msg(pf_t + qf_t[j:j + 1, :],
                               ps_t + qs_t[j:j + 1, :])
                a1 = a1 + _msg(pf_t + qf_t[j + 1:j + 2, :],
                               ps_t + qs_t[j + 1:j + 2, :])
            return (a0, a1)

        z8 = jnp.zeros((8, D), jnp.float32)
        a0, a1 = jax.lax.fori_loop(0, n_tiles, s_body, (z8, z8))
        acc = a0 + a1

        qf_d = pq_vmem[pl.ds(db, 8), D:2 * D]
        qs_d = pq_vmem[pl.ds(db, 8), 3 * D:4 * D]
        acc = acc - _msg(pf_t + qf_d, ps_t + qs_d)

        rows = jax.lax.broadcasted_iota(jnp.int32, (8, 1), 0) + db
        acc = jnp.where((rows >= rem) & (rows < rem + n),
                        acc * jnp.float32(0.5 * LN2), 0.0)

        out_ref[pl.ds(off8 + db, 8), :] += acc
        return (p1 + acc, p2 + acc * acc)

    p1, p2 = jax.lax.fori_loop(
        0, n_tiles, d_body,
        (jnp.zeros((8, D), jnp.float32), jnp.zeros((8, D), jnp.float32)))
    s1_ref[...] += jnp.sum(p1, axis=0, keepdims=True)
    s2_ref[...] += jnp.sum(p2, axis=0, keepdims=True)


def _pairwise(off, cnt, pq):
    return pl.pallas_call(
        _pair_kernel,
        grid=(N_SAMPLES,),
        in_specs=[
            pl.BlockSpec(memory_space=pltpu.SMEM),
            pl.BlockSpec(memory_space=pltpu.SMEM),
            pl.BlockSpec(memory_space=pl.ANY),
        ],
        out_specs=[
            pl.BlockSpec((N_PAD, D), lambda i: (0, 0)),
            pl.BlockSpec((1, D), lambda i: (0, 0)),
            pl.BlockSpec((1, D), lambda i: (0, 0)),
        ],
        out_shape=[
            jax.ShapeDtypeStruct((N_PAD, D), jnp.float32),
            jax.ShapeDtypeStruct((1, D), jnp.float32),
            jax.ShapeDtypeStruct((1, D), jnp.float32),
        ],
        scratch_shapes=[
            pltpu.VMEM((2, W_ROWS, 4 * D), jnp.float32),
            pltpu.SemaphoreType.DMA((2,)),
        ],
        compiler_params=pltpu.CompilerParams(
            dimension_semantics=("arbitrary",)),
    )(off, cnt, pq)


def _finalize_kernel(agg_ref, x_ref, s1_ref, s2_ref, w_ref, b_ref, o_ref):
    mean = s1_ref[...] / N_NODES
    var = s2_ref[...] / N_NODES - mean * mean
    scale = jax.lax.rsqrt(var + EPS) * w_ref[...]
    y = (agg_ref[...] - mean) * scale + b_ref[...] + x_ref[...]
    o_ref[...] = jnp.where(y >= 0, y, 0.01 * y)


def _finalize(agg, x, s1, s2, bnw, bnb):
    R = 1024
    return pl.pallas_call(
        _finalize_kernel,
        grid=(pl.cdiv(N_NODES, R),),
        in_specs=[
            pl.BlockSpec((R, D), lambda i: (i, 0)),
            pl.BlockSpec((R, D), lambda i: (i, 0)),
            pl.BlockSpec((1, D), lambda i: (0, 0)),
            pl.BlockSpec((1, D), lambda i: (0, 0)),
            pl.BlockSpec((1, D), lambda i: (0, 0)),
            pl.BlockSpec((1, D), lambda i: (0, 0)),
        ],
        out_specs=pl.BlockSpec((R, D), lambda i: (i, 0)),
        out_shape=jax.ShapeDtypeStruct((N_NODES, D), jnp.float32),
        compiler_params=pltpu.CompilerParams(
            dimension_semantics=("arbitrary",)),
    )(agg, x, s1, s2, bnw, bnb)


def kernel(gnn_in, centers, agents_per_sample, Wf1, bf1, Ws1, bs1, bnw1,
           bnb1, Wf2, bf2, Ws2, bs2, bnw2, bnb2):
    n = agents_per_sample.astype(jnp.int32)
    off = jnp.concatenate(
        [jnp.zeros((1,), jnp.int32), jnp.cumsum(n)[:-1]])

    def layer(x, Wf, bf, Ws, bs, bnw, bnb):
        cf = jnp.float32(0.5)
        cs = jnp.float32(LOG2E)
        Wcat = jnp.concatenate(
            [cf * Wf[:, :D].T, cf * Wf[:, D:2 * D].T,
             cs * Ws[:, :D].T, cs * Ws[:, D:2 * D].T], axis=1)
        We_f = Wf[:, 2 * D:].T
        We_s = Ws[:, 2 * D:].T
        Ecat = jnp.concatenate(
            [cf * We_f, -cf * We_f, cs * We_s, -cs * We_s], axis=1)
        zeros = jnp.zeros_like(bf)
        bcat = jnp.concatenate([cf * bf, zeros, cs * bs, zeros])[None, :]
        pq = _proj(x, centers, Wcat, Ecat, bcat)
        agg, s1, s2 = _pairwise(off, n, pq)
        return _finalize(agg, x, s1, s2, bnw[None], bnb[None])

    x = layer(gnn_in, Wf1, bf1, Ws1, bs1, bnw1, bnb1)
    return layer(x, Wf2, bf2, Ws2, bs2, bnw2, bnb2)


# speedup vs baseline: 1.1287x; 1.1287x over previous
import jax
import jax.numpy as jnp
from jax.experimental import pallas as pl
from jax.experimental.pallas import tpu as pltpu

N_NODES = 19719
N_SAMPLES = 313
W_ROWS = 72
N_PAD = 19776
D = 128
EPS = 1e-5
LN2 = 0.6931471805599453
LOG2E = 1.4426950408889634


NT = N_PAD // 512
RT = 512


def _layer_kernel(off_ref, cnt_ref, x_ref, c_ref, w_ref, e_ref, b_ref,
                  out_ref, s1_ref, s2_ref, pq_scr, xbuf, cbuf, xsem, csem):
    pid = pl.program_id(0)

    @pl.when(pid == 0)
    def _init():
        out_ref[...] = jnp.zeros_like(out_ref)
        s1_ref[...] = jnp.zeros_like(s1_ref)
        s2_ref[...] = jnp.zeros_like(s2_ref)
        pltpu.make_async_copy(
            x_ref.at[pl.ds(0, RT)], xbuf.at[0], xsem.at[0]).start()
        pltpu.make_async_copy(
            c_ref.at[pl.ds(0, RT)], cbuf.at[0], csem.at[0]).start()

    @pl.when(pid < NT)
    def _proj_phase():
        t = pid
        slot = jax.lax.rem(t, 2)
        nxt = jax.lax.rem(t + 1, 2)
        row = pl.multiple_of(t * RT, 8)
        pltpu.make_async_copy(
            x_ref.at[pl.ds(row, RT)], xbuf.at[slot], xsem.at[slot]).wait()
        pltpu.make_async_copy(
            c_ref.at[pl.ds(row, RT)], cbuf.at[slot], csem.at[slot]).wait()

        @pl.when(t + 1 < NT)
        def _prefetch():
            row2 = pl.multiple_of((t + 1) * RT, 8)
            pltpu.make_async_copy(
                x_ref.at[pl.ds(row2, RT)], xbuf.at[nxt], xsem.at[nxt]).start()
            pltpu.make_async_copy(
                c_ref.at[pl.ds(row2, RT)], cbuf.at[nxt], csem.at[nxt]).start()

        res = jnp.dot(xbuf[slot], w_ref[...],
                      preferred_element_type=jnp.float32)
        cw = cbuf[slot]
        res = res + cw[:, 0:1] * e_ref[0:1, :]
        res = res + cw[:, 1:2] * e_ref[1:2, :]
        pq_scr[pl.ds(row, RT), :] = res + b_ref[...]

    @pl.when(pid >= NT)
    def _pair_phase():
        k = pid - NT
        off = off_ref[k]
        n = cnt_ref[k]
        off8 = pl.multiple_of((off // 8) * 8, 8)
        rem = off - off8
        n_tiles = (rem + n + 7) // 8

        def _msg(af, as_):
            return (1.0 + jnp.tanh(af)) * jnp.log2(1.0 + jnp.exp2(as_))

        def d_body(dt, carry):
            p1, p2 = carry
            db = pl.multiple_of(off8 + dt * 8, 8)
            pf_t = pq_scr[pl.ds(db, 8), 0:D]
            ps_t = pq_scr[pl.ds(db, 8), 2 * D:3 * D]

            def s_body(st, acc):
                sb = pl.multiple_of(off8 + st * 8, 8)
                qf_t = pq_scr[pl.ds(sb, 8), D:2 * D]
                qs_t = pq_scr[pl.ds(sb, 8), 3 * D:4 * D]
                for j in range(8):
                    s_abs = sb + j
                    ok = (s_abs >= off) & (s_abs < off + n)
                    m = _msg(pf_t + qf_t[j:j + 1, :],
                             ps_t + qs_t[j:j + 1, :])
                    acc = acc + jnp.where(ok, m, 0.0)
                return acc

            acc = jax.lax.fori_loop(0, n_tiles, s_body,
                                    jnp.zeros((8, D), jnp.float32))

            qf_d = pq_scr[pl.ds(db, 8), D:2 * D]
            qs_d = pq_scr[pl.ds(db, 8), 3 * D:4 * D]
            acc = acc - _msg(pf_t + qf_d, ps_t + qs_d)

            rows = jax.lax.broadcasted_iota(jnp.int32, (8, 1), 0) + db
            acc = jnp.where((rows >= off) & (rows < off + n),
                            acc * jnp.float32(0.5 * LN2), 0.0)

            out_ref[pl.ds(db, 8), :] += acc
            return (p1 + acc, p2 + acc * acc)

        p1, p2 = jax.lax.fori_loop(
            0, n_tiles, d_body,
            (jnp.zeros((8, D), jnp.float32), jnp.zeros((8, D), jnp.float32)))
        s1_ref[...] += jnp.sum(p1, axis=0, keepdims=True)
        s2_ref[...] += jnp.sum(p2, axis=0, keepdims=True)


def _layer_call(off, cnt, x, centers, Wcat, Ecat, bcat):
    return pl.pallas_call(
        _layer_kernel,
        grid=(NT + N_SAMPLES,),
        in_specs=[
            pl.BlockSpec(memory_space=pltpu.SMEM),
            pl.BlockSpec(memory_space=pltpu.SMEM),
            pl.BlockSpec(memory_space=pl.ANY),
            pl.BlockSpec(memory_space=pl.ANY),
            pl.BlockSpec((D, 4 * D), lambda i: (0, 0)),
            pl.BlockSpec((2, 4 * D), lambda i: (0, 0)),
            pl.BlockSpec((1, 4 * D), lambda i: (0, 0)),
        ],
        out_specs=[
            pl.BlockSpec((N_PAD, D), lambda i: (0, 0)),
            pl.BlockSpec((1, D), lambda i: (0, 0)),
            pl.BlockSpec((1, D), lambda i: (0, 0)),
        ],
        out_shape=[
            jax.ShapeDtypeStruct((N_PAD, D), jnp.float32),
            jax.ShapeDtypeStruct((1, D), jnp.float32),
            jax.ShapeDtypeStruct((1, D), jnp.float32),
        ],
        scratch_shapes=[
            pltpu.VMEM((N_PAD, 4 * D), jnp.float32),
            pltpu.VMEM((2, RT, D), jnp.float32),
            pltpu.VMEM((2, RT, 2), jnp.float32),
            pltpu.SemaphoreType.DMA((2,)),
            pltpu.SemaphoreType.DMA((2,)),
        ],
        compiler_params=pltpu.CompilerParams(
            dimension_semantics=("arbitrary",)),
    )(off, cnt, x, centers, Wcat, Ecat, bcat)


def _finalize_kernel(agg_ref, x_ref, s1_ref, s2_ref, w_ref, b_ref, o_ref):
    mean = s1_ref[...] / N_NODES
    var = s2_ref[...] / N_NODES - mean * mean
    scale = jax.lax.rsqrt(var + EPS) * w_ref[...]
    y = (agg_ref[...] - mean) * scale + b_ref[...] + x_ref[...]
    o_ref[...] = jnp.where(y >= 0, y, 0.01 * y)


def _finalize(agg, x, s1, s2, bnw, bnb):
    R = 1024
    return pl.pallas_call(
        _finalize_kernel,
        grid=(pl.cdiv(N_PAD, R),),
        in_specs=[
            pl.BlockSpec((R, D), lambda i: (i, 0)),
            pl.BlockSpec((R, D), lambda i: (i, 0)),
            pl.BlockSpec((1, D), lambda i: (0, 0)),
            pl.BlockSpec((1, D), lambda i: (0, 0)),
            pl.BlockSpec((1, D), lambda i: (0, 0)),
            pl.BlockSpec((1, D), lambda i: (0, 0)),
        ],
        out_specs=pl.BlockSpec((R, D), lambda i: (i, 0)),
        out_shape=jax.ShapeDtypeStruct((N_PAD, D), jnp.float32),
        compiler_params=pltpu.CompilerParams(
            dimension_semantics=("arbitrary",)),
    )(agg, x, s1, s2, bnw, bnb)


def kernel(gnn_in, centers, agents_per_sample, Wf1, bf1, Ws1, bs1, bnw1,
           bnb1, Wf2, bf2, Ws2, bs2, bnw2, bnb2):
    n = agents_per_sample.astype(jnp.int32)
    off = jnp.concatenate(
        [jnp.zeros((1,), jnp.int32), jnp.cumsum(n)[:-1]])

    pad = ((0, N_PAD - N_NODES), (0, 0))
    x0 = jnp.pad(gnn_in, pad)
    cpad = jnp.pad(centers, pad)

    def layer(x, Wf, bf, Ws, bs, bnw, bnb):
        cf = jnp.float32(0.5)
        cs = jnp.float32(LOG2E)
        Wcat = jnp.concatenate(
            [cf * Wf[:, :D].T, cf * Wf[:, D:2 * D].T,
             cs * Ws[:, :D].T, cs * Ws[:, D:2 * D].T], axis=1)
        We_f = Wf[:, 2 * D:].T
        We_s = Ws[:, 2 * D:].T
        Ecat = jnp.concatenate(
            [cf * We_f, -cf * We_f, cs * We_s, -cs * We_s], axis=1)
        zeros = jnp.zeros_like(bf)
        bcat = jnp.concatenate([cf * bf, zeros, cs * bs, zeros])[None, :]
        agg, s1, s2 = _layer_call(off, n, x, cpad, Wcat, Ecat, bcat)
        return _finalize(agg, x, s1, s2, bnw[None], bnb[None])

    x = layer(x0, Wf1, bf1, Ws1, bs1, bnw1, bnb1)
    x = layer(x, Wf2, bf2, Ws2, bs2, bnw2, bnb2)
    return x[:N_NODES]
